# Initial kernel scaffold; baseline (speedup 1.0000x reference)
import jax
import jax.numpy as jnp
from jax.experimental import pallas as pl


def kernel(x, edge_index, edge_attr, We1, be1, We2, be2, Wn1, bn1, Wn2, bn2):
    raise NotImplementedError("write your pallas kernel here")


# speedup vs baseline: 2.8839x; 2.8839x over previous
import jax
import jax.numpy as jnp
from jax import lax
from jax.experimental import pallas as pl
from jax.experimental.pallas import tpu as pltpu
from jax.experimental.pallas import tpu_sc as plsc

N_NODES = 10000
N_EDGES = 320000
H = 128

NC = 2
NS = 16
NW = NC * NS

GC_G = 80
E_PER_W = N_EDGES // NW
G_ITERS = E_PER_W // GC_G
GC = 400
E_PER_T = N_EDGES // NS
S_ITERS = E_PER_T // GC

CNT_W = 16
HALF = 5000
A_ROWS = 5120
A_PER_T = A_ROWS // NS
DUMP_MASK = 63


def _vmesh():
    return plsc.VectorSubcoreMesh(core_axis_name="c", subcore_axis_name="s",
                                  num_cores=NC, num_subcores=NS)


def _gather_body(x_hbm, src_hbm, dst_hbm, xs_hbm, xd_hbm,
                 idx_s, idx_d, rows_s, rows_d, sem_s, sem_d):
    cid = lax.axis_index("c")
    sid = lax.axis_index("s")
    wid = cid * NS + sid

    def step(i, carry):
        base = wid * E_PER_W + i * GC_G
        pltpu.sync_copy(src_hbm.at[pl.ds(base, GC_G)], idx_s)
        pltpu.sync_copy(dst_hbm.at[pl.ds(base, GC_G)], idx_d)
        cp_s = pltpu.async_copy(x_hbm.at[idx_s], rows_s, sem_s)
        cp_d = pltpu.async_copy(x_hbm.at[idx_d], rows_d, sem_d)
        cp_s.wait()
        cp_d.wait()
        pltpu.sync_copy(rows_s, xs_hbm.at[pl.ds(base, GC_G)])
        pltpu.sync_copy(rows_d, xd_hbm.at[pl.ds(base, GC_G)])
        return carry

    lax.fori_loop(0, G_ITERS, step, None)


def _sc_gather(x, src, dst):
    return pl.kernel(
        _gather_body,
        out_type=(
            jax.ShapeDtypeStruct((N_EDGES, H), jnp.float32),
            jax.ShapeDtypeStruct((N_EDGES, H), jnp.float32),
        ),
        mesh=_vmesh(),
        scratch_types=[
            pltpu.VMEM((GC_G,), jnp.int32),
            pltpu.VMEM((GC_G,), jnp.int32),
            pltpu.VMEM((GC_G, H), jnp.float32),
            pltpu.VMEM((GC_G, H), jnp.float32),
            pltpu.SemaphoreType.DMA,
            pltpu.SemaphoreType.DMA,
        ],
    )(x, src, dst)


def _scatter_body(dst_hbm, eo_hbm, zrow_hbm, zcnt_hbm,
                  sums_hbm, cnt_hbm,
                  idx_v, idx2_v, rows_v, cnt_v, acc_sh):
    cid = lax.axis_index("c")
    sid = lax.axis_index("s")
    lo = cid * HALF

    stripe = sid * A_PER_T
    pltpu.sync_copy(zrow_hbm, rows_v.at[pl.ds(0, A_PER_T)])
    pltpu.sync_copy(rows_v.at[pl.ds(0, A_PER_T)],
                    acc_sh.at[pl.ds(stripe, A_PER_T)])
    pltpu.sync_copy(zcnt_hbm, cnt_v)
    plsc.subcore_barrier()

    def step(i, carry):
        base = sid * E_PER_T + i * GC
        pltpu.sync_copy(dst_hbm.at[pl.ds(base, GC)], idx_v)
        pltpu.sync_copy(eo_hbm.at[pl.ds(base, GC)], rows_v)
        for j in range(GC // 16):
            v = idx_v[pl.ds(j * 16, 16)]
            m = v - lo
            ok = (m >= 0) & (m < HALF)
            dump = HALF + (v & DUMP_MASK)
            v2 = jnp.where(ok, m, dump)
            idx2_v[pl.ds(j * 16, 16)] = v2
            occ, last = plsc.scan_count(v2)
            plsc.addupdate_scatter(cnt_v, [v2], occ.astype(jnp.float32),
                                   mask=last)
        pltpu.sync_copy(rows_v, acc_sh.at[idx2_v], add=True)
        return carry

    lax.fori_loop(0, S_ITERS, step, None)
    plsc.subcore_barrier()

    pltpu.sync_copy(acc_sh.at[pl.ds(stripe, A_PER_T)],
                    rows_v.at[pl.ds(0, A_PER_T)])
    pltpu.sync_copy(rows_v.at[pl.ds(0, A_PER_T)],
                    sums_hbm.at[cid, pl.ds(stripe, A_PER_T)])
    pltpu.sync_copy(cnt_v, cnt_hbm.at[cid, sid])


def _sc_scatter(edge_out, dst):
    zrow = jnp.zeros((A_PER_T, H), jnp.float32)
    zcnt = jnp.zeros((A_ROWS,), jnp.float32)
    return pl.kernel(
        _scatter_body,
        out_type=(
            jax.ShapeDtypeStruct((NC, A_ROWS, H), jnp.float32),
            jax.ShapeDtypeStruct((NC, NS, A_ROWS), jnp.float32),
        ),
        mesh=_vmesh(),
        compiler_params=pltpu.CompilerParams(needs_layout_passes=False),
        scratch_types=[
            pltpu.VMEM((GC,), jnp.int32),
            pltpu.VMEM((GC,), jnp.int32),
            pltpu.VMEM((GC, H), jnp.float32),
            pltpu.VMEM((A_ROWS,), jnp.float32),
            pltpu.VMEM_SHARED((A_ROWS, H), jnp.float32),
        ],
    )(dst, edge_out, zrow, zcnt)


def _elu(z):
    return jnp.where(z > 0, z, jnp.exp(jnp.minimum(z, 0.0)) - 1.0)


def _edge_mlp_body(ea_ref, xs_ref, xd_ref, wa_ref, ws_ref, wd_ref, b1_ref,
                   w2_ref, b2_ref, out_ref):
    ea = ea_ref[...]
    z = (jnp.dot(ea, wa_ref[...], preferred_element_type=jnp.float32)
         + jnp.dot(xs_ref[...], ws_ref[...], preferred_element_type=jnp.float32)
         + jnp.dot(xd_ref[...], wd_ref[...], preferred_element_type=jnp.float32)
         + b1_ref[...])
    h = _elu(z)
    out_ref[...] = ea + jnp.dot(h, w2_ref[...],
                                preferred_element_type=jnp.float32) + b2_ref[...]


def _tc_edge_mlp(edge_attr, xs, xd, We1, be1, We2, be2):
    EB = 2000
    grid = (N_EDGES // EB,)
    row_spec = pl.BlockSpec((EB, H), lambda i: (i, 0))
    w_spec = pl.BlockSpec((H, H), lambda i: (0, 0))
    b_spec = pl.BlockSpec((1, H), lambda i: (0, 0))
    return pl.pallas_call(
        _edge_mlp_body,
        grid=grid,
        in_specs=[row_spec, row_spec, row_spec,
                  w_spec, w_spec, w_spec, b_spec, w_spec, b_spec],
        out_specs=row_spec,
        out_shape=jax.ShapeDtypeStruct((N_EDGES, H), jnp.float32),
        compiler_params=pltpu.CompilerParams(
            dimension_semantics=("arbitrary",)),
    )(edge_attr, xs, xd, We1[:H], We1[H:2 * H], We1[2 * H:],
      be1.reshape(1, H), We2, be2.reshape(1, H))


def _node_mlp_body(x_ref, s_ref, c_ref,
                   wx_ref, wg_ref, b1_ref, w2_ref, b2_ref, out_ref):
    x = x_ref[...]
    ones_col = jnp.ones((NS, 1), jnp.float32)
    cnt = lax.dot_general(c_ref[0], ones_col, (((0,), (0,)), ((), ())),
                          preferred_element_type=jnp.float32)[:x.shape[0]]
    agg = s_ref[0] / jnp.maximum(cnt, 1.0)
    z = (jnp.dot(x, wx_ref[...], preferred_element_type=jnp.float32)
         + jnp.dot(agg, wg_ref[...], preferred_element_type=jnp.float32)
         + b1_ref[...])
    h = _elu(z)
    out_ref[...] = x + jnp.dot(h, w2_ref[...],
                               preferred_element_type=jnp.float32) + b2_ref[...]


def _tc_node_mlp(x, sums_p, cnt_p, Wn1, bn1, Wn2, bn2):
    NB = HALF
    grid = (N_NODES // NB,)
    row_spec = pl.BlockSpec((NB, H), lambda i: (i, 0))
    sum_spec = pl.BlockSpec((1, NB, H), lambda i: (i, 0, 0))
    cnt_spec = pl.BlockSpec((1, NS, A_ROWS), lambda i: (i, 0, 0))
    w_spec = pl.BlockSpec((H, H), lambda i: (0, 0))
    b_spec = pl.BlockSpec((1, H), lambda i: (0, 0))
    return pl.pallas_call(
        _node_mlp_body,
        grid=grid,
        in_specs=[row_spec, sum_spec, cnt_spec,
                  w_spec, w_spec, b_spec, w_spec, b_spec],
        out_specs=row_spec,
        out_shape=jax.ShapeDtypeStruct((N_NODES, H), jnp.float32),
        compiler_params=pltpu.CompilerParams(
            dimension_semantics=("arbitrary",)),
    )(x, sums_p, cnt_p,
      Wn1[:H], Wn1[H:], bn1.reshape(1, H), Wn2, bn2.reshape(1, H))


def kernel(x, edge_index, edge_attr, We1, be1, We2, be2, Wn1, bn1, Wn2, bn2):
    src = edge_index[0]
    dst = edge_index[1]
    xs, xd = _sc_gather(x, src, dst)
    edge_out = _tc_edge_mlp(edge_attr, xs, xd, We1, be1, We2, be2)
    sums_p, cnt_p = _sc_scatter(edge_out, dst)
    node_out = _tc_node_mlp(x, sums_p, cnt_p, Wn1, bn1, Wn2, bn2)
    return node_out, edge_out
